# baseline (speedup 1.0000x reference)
import functools

import jax
import jax.numpy as jnp
from jax import lax
from jax.experimental import pallas as pl
from jax.experimental.pallas import tpu as pltpu
from jax.experimental.pallas import tpu_sc as plsc

NC, NS = 2, 16
NW = NC * NS
CH = 400
NBUF = 4


def kernel(x, weight):
    S0, S1 = x.shape
    V, D = weight.shape
    B = S0 * S1
    assert B % (NW * CH * NBUF) == 0
    b_per_w = B // NW
    n_chunks = b_per_w // CH
    n_groups = n_chunks // NBUF

    xf = x.reshape(B)

    mesh = plsc.VectorSubcoreMesh(core_axis_name="c", subcore_axis_name="s")

    @functools.partial(
        pl.kernel,
        out_type=jax.ShapeDtypeStruct((B, 2 * D), jnp.float32),
        mesh=mesh,
        scratch_types=(
            [pltpu.VMEM((b_per_w,), jnp.int32)]
            + [pltpu.VMEM((CH, D), jnp.float32) for _ in range(NBUF)]
            + [pltpu.SemaphoreType.DMA for _ in range(2 * NBUF)]
        ),
        compiler_params=pltpu.CompilerParams(use_tc_tiling_on_sc=False),
    )
    def emb(idx_hbm, table_hbm, out_hbm, idx_all, *bufs_and_sems):
        rows = bufs_and_sems[:NBUF]
        gsem = bufs_and_sems[NBUF:2 * NBUF]
        osem = bufs_and_sems[2 * NBUF:]
        wid = lax.axis_index("s") * NC + lax.axis_index("c")
        base = wid * b_per_w

        pltpu.sync_copy(idx_hbm.at[pl.ds(base, b_per_w)], idx_all)

        def start_gather(p, slot):
            pltpu.async_copy(
                table_hbm.at[idx_all.at[pl.ds(p * CH, CH)]], rows[slot], gsem[slot]
            )

        def wait_gather(slot):
            pltpu.make_async_copy(
                out_hbm.at[pl.ds(0, CH), pl.ds(0, D)], rows[slot], gsem[slot]
            ).wait()

        def start_write(g, slot):
            pltpu.async_copy(
                rows[slot],
                out_hbm.at[pl.ds(base + g * CH, CH), pl.ds(0, D)],
                osem[slot],
            )

        def wait_write(slot):
            pltpu.make_async_copy(
                rows[slot], out_hbm.at[pl.ds(0, CH), pl.ds(0, D)], osem[slot]
            ).wait()

        start_gather(0, 0)
        start_gather(1, 1)

        def group(i, carry):
            gbase = i * NBUF
            for b in range(NBUF):
                g = gbase + b
                wait_gather(b)
                start_write(g, b)
                sp = (b + 2) % NBUF

                @pl.when(g + 2 < n_chunks)
                def _prefetch():
                    @pl.when(g >= 2)
                    def _drain():
                        wait_write(sp)

                    start_gather(g + 2, sp)

            return carry

        lax.fori_loop(0, n_groups, group, 0)
        for b in range(NBUF):
            wait_write(b)

    out = emb(xf, weight)
    return out.reshape(S0, S1, 2 * D)[:, :, :D]
